# Initial kernel scaffold; baseline (speedup 1.0000x reference)
import jax
import jax.numpy as jnp
from jax.experimental import pallas as pl


def kernel(x, weight):
    raise NotImplementedError("write your pallas kernel here")


# speedup vs baseline: 4.1572x; 4.1572x over previous
import jax
import jax.numpy as jnp
from jax.experimental import pallas as pl
from jax.experimental.pallas import tpu as pltpu

OUT_F = 512
IN_F = 784
VTHR = 12500.0
PROHIB = 11250.0


def _snn_step(x_ref, w_ref, out_ref, W_s, mem_s, cin_s, cout_s, any_s):
    t = pl.program_id(0)

    @pl.when(t == 0)
    def _init():
        W_s[...] = w_ref[...]
        mem_s[...] = jnp.zeros_like(mem_s)
        cin_s[...] = jnp.zeros_like(cin_s)
        cout_s[...] = jnp.zeros_like(cout_s)
        any_s[0] = 0.0

    ind = x_ref[0]
    W = W_s[...]

    prohibit = jnp.where(any_s[0] > 0.0, PROHIB, 0.0)
    psum = jax.lax.dot_general(
        W, ind, (((1,), (1,)), ((), ())), preferred_element_type=jnp.float32
    )
    mem = jnp.maximum(mem_s[...] + psum - prohibit, 0.0)
    spike = mem >= VTHR
    spike_f = spike.astype(jnp.float32)
    mem_s[...] = jnp.where(spike, 0.0, mem)
    out_ref[0] = spike_f
    any_s[0] = jnp.sum(spike_f)

    cin = jnp.where(ind == 1.0, 0.0, jnp.minimum(cin_s[...] + 1.0, 31.0))
    cin_s[...] = cin
    cout = jnp.where(spike, 0.0, jnp.minimum(cout_s[...] + 1.0, 31.0))
    cout_s[...] = cout

    a = jnp.where(cin == 0.0, 2.0, jnp.where(cin == 1.0, 1.0, 0.0))
    b = jnp.where(jnp.logical_and(jnp.logical_not(spike), cout == 1.0),
                  -1.0, 0.0)
    W_s[...] = jnp.clip(W + spike_f * a + b * ind, 0.0, 127.0)


def _run(x3d, weight, interpret=False):
    T = x3d.shape[0]
    spikes_cols = pl.pallas_call(
        _snn_step,
        grid=(T,),
        in_specs=[
            pl.BlockSpec((1, 1, IN_F), lambda t: (t, 0, 0)),
            pl.BlockSpec((OUT_F, IN_F), lambda t: (0, 0)),
        ],
        out_specs=pl.BlockSpec((1, OUT_F, 1), lambda t: (t, 0, 0)),
        out_shape=jax.ShapeDtypeStruct((T, OUT_F, 1), jnp.float32),
        scratch_shapes=[
            pltpu.VMEM((OUT_F, IN_F), jnp.float32),
            pltpu.VMEM((OUT_F, 1), jnp.float32),
            pltpu.VMEM((1, IN_F), jnp.float32),
            pltpu.VMEM((OUT_F, 1), jnp.float32),
            pltpu.SMEM((1,), jnp.float32),
        ],
        compiler_params=pltpu.CompilerParams(
            dimension_semantics=("arbitrary",),
        ),
        interpret=interpret,
    )(x3d, weight)
    return spikes_cols


def kernel(x, weight):
    T = x.shape[0]
    spikes = _run(x, weight)
    return spikes.reshape(T, 1, OUT_F)
